# baseline (speedup 1.0000x reference)
import jax
import jax.numpy as jnp
from jax.experimental import pallas as pl
from jax.experimental.pallas import tpu as pltpu

_N = 100000
_BLK = 4000


def _fused_body(x_ref,
                w0, w1, w2, w3, w4, w5, w6, w7, w8,
                b0, b1, b2, b3, b4, b5, b6, b7, b8,
                s0, s1, s2, t0, t1, t2,
                nodes_ref, sum_ref):
    i = pl.program_id(0)
    ws = (w0, w1, w2, w3, w4, w5, w6, w7, w8)
    bs = (b0, b1, b2, b3, b4, b5, b6, b7, b8)
    lns = ((s0, t0), (s1, t1), (s2, t2))

    h = x_ref[...]
    for step in range(3):
        for layer in range(3):
            k = 3 * step + layer
            h = jnp.dot(h, ws[k][...], preferred_element_type=jnp.float32)
            h = h + bs[k][...]
            if layer < 2:
                h = jax.nn.gelu(h)
        scale, bias = lns[step]
        mu = jnp.mean(h, axis=-1, keepdims=True)
        var = jnp.mean(jnp.square(h - mu), axis=-1, keepdims=True)
        h = (h - mu) * jax.lax.rsqrt(var + 1e-6) * scale[...] + bias[...]

    nodes_ref[...] = h
    blk_sum = jnp.sum(h, axis=0, keepdims=True)

    @pl.when(i == 0)
    def _init():
        sum_ref[...] = blk_sum

    @pl.when(i > 0)
    def _acc():
        sum_ref[...] += blk_sum


def _readout_body(sum_ref, rw0, rb0, rw1, rb1, rw2t, rb2, out_ref):
    g = sum_ref[...] * (1.0 / _N)
    h = jax.nn.gelu(jnp.dot(g, rw0[...], preferred_element_type=jnp.float32) + rb0[...])
    h = jax.nn.gelu(jnp.dot(h, rw1[...], preferred_element_type=jnp.float32) + rb1[...])
    o = jnp.sum(h * rw2t[...], axis=-1, keepdims=True) + rb2[...]
    out_ref[...] = o


def _full(shape):
    return pl.BlockSpec(shape, lambda *a: tuple(0 for _ in shape))


def kernel(x, segment_ids, params):
    del segment_ids
    steps = params["steps"]
    ws = [steps[s]["node_mlp"][l]["w"] for s in range(3) for l in range(3)]
    bs = [steps[s]["node_mlp"][l]["b"].reshape(1, -1) for s in range(3) for l in range(3)]
    lss = [steps[s]["ln"]["scale"].reshape(1, -1) for s in range(3)]
    lbs = [steps[s]["ln"]["bias"].reshape(1, -1) for s in range(3)]

    grid = _N // _BLK
    in_specs = (
        [pl.BlockSpec((_BLK, 128), lambda i: (i, 0))]
        + [_full(w.shape) for w in ws]
        + [_full(b.shape) for b in bs]
        + [_full(s.shape) for s in lss]
        + [_full(b.shape) for b in lbs]
    )
    nodes, tot = pl.pallas_call(
        _fused_body,
        grid=(grid,),
        in_specs=in_specs,
        out_specs=(
            pl.BlockSpec((_BLK, 64), lambda i: (i, 0)),
            pl.BlockSpec((1, 64), lambda i: (0, 0)),
        ),
        out_shape=(
            jax.ShapeDtypeStruct((_N, 64), jnp.float32),
            jax.ShapeDtypeStruct((1, 64), jnp.float32),
        ),
        compiler_params=pltpu.CompilerParams(
            dimension_semantics=("arbitrary",),
        ),
    )(x, *ws, *bs, *lss, *lbs)

    ro = params["readout"]
    rw0, rb0 = ro[0]["w"], ro[0]["b"].reshape(1, -1)
    rw1, rb1 = ro[1]["w"], ro[1]["b"].reshape(1, -1)
    rw2t, rb2 = ro[2]["w"].reshape(1, -1), ro[2]["b"].reshape(1, -1)

    out = pl.pallas_call(
        _readout_body,
        in_specs=[_full(a.shape) for a in (tot, rw0, rb0, rw1, rb1, rw2t, rb2)],
        out_specs=pl.BlockSpec((1, 1), lambda: (0, 0)),
        out_shape=jax.ShapeDtypeStruct((1, 1), jnp.float32),
    )(tot, rw0, rb0, rw1, rb1, rw2t, rb2)

    return (out.reshape(1), nodes)
